# baseline (speedup 1.0000x reference)
import functools

import jax
import jax.numpy as jnp
from jax import lax
from jax.experimental import pallas as pl
from jax.experimental.pallas import tpu as pltpu
from jax.experimental.pallas import tpu_sc as plsc

D = 64
CHUNK = 100


@functools.lru_cache(maxsize=None)
def _make_sc_kernel(B, L, NC, NS):
    NW = NC * NS
    b_per_w = B // NW
    n_chunks = b_per_w * (L // CHUNK)
    NBUF = 4
    n_groups = n_chunks // NBUF
    mesh = plsc.VectorSubcoreMesh(core_axis_name="c", subcore_axis_name="s")

    @functools.partial(
        pl.kernel,
        out_type=jax.ShapeDtypeStruct((B, L, D), jnp.float32),
        mesh=mesh,
        compiler_params=pltpu.CompilerParams(use_tc_tiling_on_sc=False),
        scratch_types=[
            pltpu.VMEM((n_chunks, CHUNK), jnp.int32),
            pltpu.VMEM((n_chunks, CHUNK), jnp.int32),
            pltpu.VMEM((n_chunks, CHUNK), jnp.int32),
            [pltpu.VMEM((CHUNK, D), jnp.float32)] * NBUF,
            [pltpu.SemaphoreType.DMA] * NBUF,
            [pltpu.SemaphoreType.DMA] * NBUF,
            [pltpu.SemaphoreType.DMA] * NBUF,
        ],
    )
    def body(data_h, shift_h, total_h, wd_h, ws_h, wt_h, out_h,
             idx_d, idx_s, idx_t, accs, gsems, asems, ssems):
        wid = lax.axis_index("s") * NC + lax.axis_index("c")
        b0 = wid * b_per_w
        pltpu.sync_copy(data_h.at[wid], idx_d)
        pltpu.sync_copy(shift_h.at[wid], idx_s)
        pltpu.sync_copy(total_h.at[wid], idx_t)

        def dst_slice(c):
            return out_h.at[b0 + c // 2, pl.ds((c % 2) * CHUNK, CHUNK), :]

        def group_body(g, carry):
            for b in range(NBUF):
                c = g * NBUF + b

                @pl.when(g > 0)
                def _wait_prev_store(b=b, c=c):
                    pltpu.make_async_copy(accs[b], dst_slice(c - NBUF),
                                          ssems[b]).wait()

                pltpu.async_copy(wd_h.at[idx_d.at[c]], accs[b], gsems[b])
            for b in range(NBUF):
                c = g * NBUF + b
                pltpu.make_async_copy(wd_h.at[idx_d.at[c]], accs[b],
                                      gsems[b]).wait()
                pltpu.async_copy(ws_h.at[idx_s.at[c]], accs[b], asems[b],
                                 add=True)
                pltpu.async_copy(wt_h.at[idx_t.at[c]], accs[b], asems[b],
                                 add=True)
            for b in range(NBUF):
                c = g * NBUF + b
                add_cp = pltpu.make_async_copy(ws_h.at[idx_s.at[c]], accs[b],
                                               asems[b])
                add_cp.wait()
                add_cp.wait()
                pltpu.async_copy(accs[b], dst_slice(c), ssems[b])
            return carry

        lax.fori_loop(0, n_groups, group_body, 0)
        for b in range(NBUF):
            c = (n_groups - 1) * NBUF + b
            pltpu.make_async_copy(accs[b], dst_slice(c), ssems[b]).wait()

    return body


def kernel(data, shift, total, W_data, W_shift, W_total):
    B, L = data.shape
    info = plsc.get_sparse_core_info()
    NC, NS = info.num_cores, info.num_subcores
    NW = NC * NS
    n_chunks = (B // NW) * (L // CHUNK)
    d3 = data.reshape(NW, n_chunks, CHUNK).astype(jnp.int32)
    s3 = shift.reshape(NW, n_chunks, CHUNK).astype(jnp.int32)
    t3 = total.reshape(NW, n_chunks, CHUNK).astype(jnp.int32)
    return _make_sc_kernel(B, L, NC, NS)(
        d3, s3, t3, W_data, W_shift, W_total)
